# baseline (speedup 1.0000x reference)
import functools
import jax
import jax.numpy as jnp
from jax import lax
from jax.experimental import pallas as pl
from jax.experimental.pallas import tpu as pltpu
from jax.experimental.pallas import tpu_sc as plsc

PAD = 4
L = 16
W = 32
DEPTH = 6


@functools.lru_cache(maxsize=None)
def _make_lookup(V, D, S, Bb):
    CH = 2048
    NB = Bb // CH
    CPP = S * NB
    PW = (D * CPP) // W
    assert Bb % CH == 0 and CH % L == 0 and D * CPP == W * PW
    for w in range(W):
        r0 = (w * PW) % CPP
        l0 = min(CPP - r0, PW)
        assert l0 >= DEPTH and (PW - l0 == 0 or PW - l0 >= DEPTH)

    mesh = plsc.VectorSubcoreMesh(core_axis_name="c", subcore_axis_name="s")

    @functools.partial(
        pl.kernel,
        out_type=jax.ShapeDtypeStruct((D, S, Bb), jnp.float32),
        mesh=mesh,
        compiler_params=pltpu.CompilerParams(
            use_tc_tiling_on_sc=True, needs_layout_passes=False
        ),
        scratch_types=(
            [pltpu.VMEM((V,), jnp.float32)]
            + [pltpu.VMEM((CH,), jnp.int32)] * DEPTH
            + [pltpu.VMEM((CH,), jnp.float32)] * DEPTH
            + [pltpu.SemaphoreType.DMA] * (2 * DEPTH)
        ),
    )
    def lookup(tT_hbm, xT_hbm, out_hbm, plane_v, *bufs):
        ibufs = bufs[0:DEPTH]
        obufs = bufs[DEPTH:2 * DEPTH]
        isems = bufs[2 * DEPTH:3 * DEPTH]
        osems = bufs[3 * DEPTH:4 * DEPTH]

        wid = lax.axis_index("s") * 2 + lax.axis_index("c")
        start = wid * PW
        d0 = start // CPP
        r0 = start % CPP
        len0 = jnp.minimum(CPP - r0, PW)
        len1 = PW - len0

        def run_segment(d, c0, steps):
            pltpu.sync_copy(tT_hbm.at[d], plane_v)
            lane = lax.iota(jnp.int32, L)
            plane_v[pl.ds(0, L)] = jnp.where(
                lane == PAD, 0.0, plane_v[pl.ds(0, L)]
            )

            def start_idx(j, b):
                c = c0 + j
                srow = c // NB
                bcol = (c % NB) * CH
                pltpu.make_async_copy(
                    xT_hbm.at[srow, pl.ds(bcol, CH)], ibufs[b], isems[b]
                ).start()

            def wait_idx(b):
                pltpu.make_async_copy(
                    xT_hbm.at[0, pl.ds(0, CH)], ibufs[b], isems[b]
                ).wait()

            def start_out(j, b):
                c = c0 + j
                srow = c // NB
                bcol = (c % NB) * CH
                pltpu.make_async_copy(
                    obufs[b], out_hbm.at[d, srow, pl.ds(bcol, CH)], osems[b]
                ).start()

            def wait_out(b):
                pltpu.make_async_copy(
                    obufs[b], out_hbm.at[d, 0, pl.ds(0, CH)], osems[b]
                ).wait()

            for b in range(DEPTH):
                start_idx(b, b)

            def body(j):
                for b in range(DEPTH):
                    jj = j + b

                    @pl.when(jj < steps)
                    def _():
                        wait_idx(b)

                        @pl.when(jj >= DEPTH)
                        def _():
                            wait_out(b)

                        def grp(g):
                            for u in range(16):
                                off = (g + u) * L
                                iv = ibufs[b][pl.ds(off, L)]
                                obufs[b][pl.ds(off, L)] = plsc.load_gather(
                                    plane_v, [iv]
                                )

                        pl.loop(0, CH // L, step=16)(grp)
                        start_out(jj, b)

                        @pl.when(jj + DEPTH < steps)
                        def _():
                            start_idx(jj + DEPTH, b)

            pl.loop(0, steps, step=DEPTH)(body)
            for b in range(DEPTH):
                wait_out(b)

        run_segment(d0, r0, len0)

        @pl.when(len1 > 0)
        def _():
            run_segment(d0 + 1, jnp.int32(0), len1)

    return lookup


def kernel(x, table):
    B_, S = x.shape
    V, D = table.shape
    out3 = _make_lookup(V, D, S, B_)(table.T, x.T)
    return jnp.transpose(out3, (2, 1, 0))


# speedup vs baseline: 1.0263x; 1.0263x over previous
import functools
import jax
import jax.numpy as jnp
from jax import lax
from jax.experimental import pallas as pl
from jax.experimental.pallas import tpu as pltpu
from jax.experimental.pallas import tpu_sc as plsc

PAD = 4
L = 16
W = 32
DEPTH = 3


@functools.lru_cache(maxsize=None)
def _make_lookup(V, D, S, Bb):
    CH = 4096
    NB = Bb // CH
    CPP = S * NB
    PW = (D * CPP) // W
    assert Bb % CH == 0 and CH % L == 0 and D * CPP == W * PW
    for w in range(W):
        r0 = (w * PW) % CPP
        l0 = min(CPP - r0, PW)
        assert l0 >= DEPTH and (PW - l0 == 0 or PW - l0 >= DEPTH)

    mesh = plsc.VectorSubcoreMesh(core_axis_name="c", subcore_axis_name="s")

    @functools.partial(
        pl.kernel,
        out_type=jax.ShapeDtypeStruct((D, S, Bb), jnp.float32),
        mesh=mesh,
        compiler_params=pltpu.CompilerParams(
            use_tc_tiling_on_sc=True, needs_layout_passes=False
        ),
        scratch_types=(
            [pltpu.VMEM((V,), jnp.float32)]
            + [pltpu.VMEM((CH,), jnp.int32)] * DEPTH
            + [pltpu.VMEM((CH,), jnp.float32)] * DEPTH
            + [pltpu.SemaphoreType.DMA] * (2 * DEPTH)
        ),
    )
    def lookup(tT_hbm, xT_hbm, out_hbm, plane_v, *bufs):
        ibufs = bufs[0:DEPTH]
        obufs = bufs[DEPTH:2 * DEPTH]
        isems = bufs[2 * DEPTH:3 * DEPTH]
        osems = bufs[3 * DEPTH:4 * DEPTH]

        wid = lax.axis_index("s") * 2 + lax.axis_index("c")
        start = wid * PW
        d0 = start // CPP
        r0 = start % CPP
        len0 = jnp.minimum(CPP - r0, PW)
        len1 = PW - len0

        def run_segment(d, c0, steps):
            pltpu.sync_copy(tT_hbm.at[d], plane_v)
            lane = lax.iota(jnp.int32, L)
            plane_v[pl.ds(0, L)] = jnp.where(
                lane == PAD, 0.0, plane_v[pl.ds(0, L)]
            )

            def start_idx(j, b):
                c = c0 + j
                srow = c // NB
                bcol = (c % NB) * CH
                pltpu.make_async_copy(
                    xT_hbm.at[srow, pl.ds(bcol, CH)], ibufs[b], isems[b]
                ).start()

            def wait_idx(b):
                pltpu.make_async_copy(
                    xT_hbm.at[0, pl.ds(0, CH)], ibufs[b], isems[b]
                ).wait()

            def start_out(j, b):
                c = c0 + j
                srow = c // NB
                bcol = (c % NB) * CH
                pltpu.make_async_copy(
                    obufs[b], out_hbm.at[d, srow, pl.ds(bcol, CH)], osems[b]
                ).start()

            def wait_out(b):
                pltpu.make_async_copy(
                    obufs[b], out_hbm.at[d, 0, pl.ds(0, CH)], osems[b]
                ).wait()

            for b in range(DEPTH):
                start_idx(b, b)

            def body(j):
                for b in range(DEPTH):
                    jj = j + b

                    @pl.when(jj < steps)
                    def _():
                        wait_idx(b)

                        @pl.when(jj >= DEPTH)
                        def _():
                            wait_out(b)

                        def grp(g):
                            for u in range(16):
                                off = (g + u) * L
                                iv = ibufs[b][pl.ds(off, L)]
                                obufs[b][pl.ds(off, L)] = plsc.load_gather(
                                    plane_v, [iv]
                                )

                        pl.loop(0, CH // L, step=16)(grp)
                        start_out(jj, b)

                        @pl.when(jj + DEPTH < steps)
                        def _():
                            start_idx(jj + DEPTH, b)

            pl.loop(0, steps, step=DEPTH)(body)
            for b in range(DEPTH):
                wait_out(b)

        run_segment(d0, r0, len0)

        @pl.when(len1 > 0)
        def _():
            run_segment(d0 + 1, jnp.int32(0), len1)

    return lookup


def kernel(x, table):
    B_, S = x.shape
    V, D = table.shape
    out3 = _make_lookup(V, D, S, B_)(table.T, x.T)
    return jnp.transpose(out3, (2, 1, 0))
